# baseline (speedup 1.0000x reference)
import functools

import jax
import jax.numpy as jnp
from jax import lax
from jax.experimental import pallas as pl
from jax.experimental.pallas import tpu as pltpu
from jax.experimental.pallas import tpu_sc as plsc

NC, NS, LANES = 2, 16, 16
NW = NC * NS
EMB = 128
CHUNK = 128
FSTEPS = EMB // LANES
NBUF = 3


def _sc_mesh():
    return plsc.VectorSubcoreMesh(core_axis_name="c", subcore_axis_name="s",
                                  num_cores=NC, num_subcores=NS)


def _sc_aggregate(y, idx3, wgt3, dst3, n_pad, n_chunks):
    n_half = n_pad // 2
    acc_rows = n_half + CHUNK
    zslice = acc_rows // NS
    wslice = n_half // NS

    @functools.partial(
        pl.kernel,
        out_type=jax.ShapeDtypeStruct((NC * n_pad, EMB), jnp.float32),
        mesh=_sc_mesh(),
        scratch_types=[
            pltpu.VMEM((n_chunks, CHUNK), jnp.int32),
            pltpu.VMEM((n_chunks, CHUNK), jnp.float32),
            pltpu.VMEM((n_chunks, CHUNK), jnp.int32),
            [pltpu.VMEM((CHUNK,), jnp.int32) for _ in range(NBUF)],
            [pltpu.VMEM((CHUNK, EMB), jnp.float32) for _ in range(NBUF)],
            pltpu.VMEM_SHARED((acc_rows, EMB), jnp.float32),
            [pltpu.SemaphoreType.DMA for _ in range(NBUF)],
            [pltpu.SemaphoreType.DMA for _ in range(NBUF)],
        ],
        compiler_params=pltpu.CompilerParams(needs_layout_passes=False),
    )
    def k(y_hbm, idx_hbm, wgt_hbm, dst_hbm, out_hbm,
          idx_v, wgt_v, dst_v, dst_locs, bufs, acc, sem_g, sem_s):
        cid = lax.axis_index("c")
        sid = lax.axis_index("s")
        tid = cid * NS + sid

        pltpu.sync_copy(idx_hbm.at[tid], idx_v)
        pltpu.sync_copy(wgt_hbm.at[tid], wgt_v)
        pltpu.sync_copy(dst_hbm.at[tid], dst_v)

        zeros = jnp.zeros((LANES,), jnp.float32)

        def start_gather(b, j):
            pltpu.async_copy(y_hbm.at[idx_v.at[j]], bufs[b], sem_g[b])

        def wait_gather(b):
            pltpu.make_async_copy(y_hbm.at[idx_v.at[0]], bufs[b],
                                  sem_g[b]).wait()

        def start_scatter(b):
            pltpu.async_copy(bufs[b], acc.at[dst_locs[b]], sem_s[b], add=True)

        def wait_scatter(b):
            pltpu.make_async_copy(bufs[b], acc.at[dst_locs[b]],
                                  sem_s[b]).wait()

        def scale(b, j):
            buf = bufs[b]
            jv = jnp.full((LANES,), j, jnp.int32)

            @pl.loop(0, CHUNK, unroll=4)
            def _(e):
                w = plsc.load_gather(wgt_v, [jv, jnp.full((LANES,), e, jnp.int32)])
                for f in range(FSTEPS):
                    sl = pl.ds(f * LANES, LANES)
                    buf[e, sl] = buf[e, sl] * w

        last = n_chunks - 1

        for h in (0, 1):
            plsc.subcore_barrier()

            @pl.loop(0, CHUNK)
            def _(e):
                for f in range(FSTEPS):
                    bufs[0][e, pl.ds(f * LANES, LANES)] = zeros

            z0 = sid * zslice
            for off, size in ((0, CHUNK), (CHUNK, CHUNK),
                              (2 * CHUNK, zslice - 2 * CHUNK)):
                pltpu.sync_copy(bufs[0].at[pl.ds(0, size)],
                                acc.at[pl.ds(z0 + off, size)])
            plsc.subcore_barrier()

            def make_dst(b, j):
                for q in range(CHUNK // LANES):
                    sl = pl.ds(q * LANES, LANES)
                    d = dst_v[j, sl]
                    local = d - h * n_half
                    ok = (local >= 0) & (local < n_half)
                    dst_locs[b][sl] = jnp.where(ok, local,
                                                n_half + (d & (CHUNK - 1)))

            for b in range(NBUF):
                start_gather(b, b)

            @pl.loop(0, n_chunks, step=NBUF)
            def _(g):
                for b in range(NBUF):
                    wait_gather(b)
                    scale(b, g + b)
                    make_dst(b, g + b)
                    pltpu.sync_copy(bufs[b], acc.at[dst_locs[b]], add=True)
                    start_gather(b, jnp.minimum(g + NBUF + b, last))

            for b in range(NBUF):
                wait_gather(b)
            plsc.subcore_barrier()

            for off, size in ((0, CHUNK), (CHUNK, CHUNK),
                              (2 * CHUNK, wslice - 2 * CHUNK)):
                r0 = sid * wslice + off
                pltpu.sync_copy(acc.at[pl.ds(r0, size)],
                                bufs[0].at[pl.ds(0, size)])
                pltpu.sync_copy(
                    bufs[0].at[pl.ds(0, size)],
                    out_hbm.at[pl.ds(cid * n_pad + h * n_half + r0, size)])

    return k(y, idx3, wgt3, dst3)


def _sc_gather_rows(x, idx3, n_rows_per_tile):
    nch = n_rows_per_tile // CHUNK
    n_out = NW * n_rows_per_tile

    @functools.partial(
        pl.kernel,
        out_type=jax.ShapeDtypeStruct((n_out, EMB), jnp.float32),
        mesh=_sc_mesh(),
        scratch_types=[
            pltpu.VMEM((nch, CHUNK), jnp.int32),
            pltpu.VMEM((CHUNK, EMB), jnp.float32),
            pltpu.VMEM((CHUNK, EMB), jnp.float32),
            pltpu.SemaphoreType.DMA,
            pltpu.SemaphoreType.DMA,
        ],
    )
    def k(x_hbm, idx_hbm, out_hbm, idx_v, buf_a, buf_b, sem_a, sem_b):
        cid = lax.axis_index("c")
        sid = lax.axis_index("s")
        tid = cid * NS + sid
        base = tid * n_rows_per_tile

        pltpu.sync_copy(idx_hbm.at[tid], idx_v)

        def start_gather(buf, sem, j):
            pltpu.async_copy(x_hbm.at[idx_v.at[j]], buf, sem)

        def wait_gather(buf, sem):
            pltpu.make_async_copy(x_hbm.at[idx_v.at[0]], buf, sem).wait()

        last = nch - 1
        start_gather(buf_a, sem_a, 0)

        @pl.loop(0, nch, step=2)
        def _(j):
            wait_gather(buf_a, sem_a)
            start_gather(buf_b, sem_b, j + 1)
            pltpu.sync_copy(buf_a, out_hbm.at[pl.ds(base + j * CHUNK, CHUNK)])
            wait_gather(buf_b, sem_b)
            start_gather(buf_a, sem_a, jnp.minimum(j + 2, last))
            pltpu.sync_copy(buf_b,
                            out_hbm.at[pl.ds(base + (j + 1) * CHUNK, CHUNK)])

        wait_gather(buf_a, sem_a)

    return k(x, idx3)


_TC_BLK = 1000


def _tc_first(x, w3, bias):
    n = x.shape[0]

    def body(x_ref, w_ref, b_ref, y_ref, r_ref):
        xb = x_ref[...]
        y_ref[0] = jnp.dot(xb, w_ref[0], preferred_element_type=jnp.float32)
        y_ref[1] = jnp.dot(xb, w_ref[1], preferred_element_type=jnp.float32)
        r_ref[...] = (jnp.dot(xb, w_ref[2], preferred_element_type=jnp.float32)
                      + b_ref[...])

    return pl.pallas_call(
        body,
        grid=(n // _TC_BLK,),
        in_specs=[
            pl.BlockSpec((_TC_BLK, EMB), lambda i: (i, 0)),
            pl.BlockSpec((3, EMB, EMB), lambda i: (0, 0, 0)),
            pl.BlockSpec((1, EMB), lambda i: (0, 0)),
        ],
        out_specs=[
            pl.BlockSpec((2, _TC_BLK, EMB), lambda i: (0, i, 0)),
            pl.BlockSpec((_TC_BLK, EMB), lambda i: (i, 0)),
        ],
        out_shape=[
            jax.ShapeDtypeStruct((2, n, EMB), jnp.float32),
            jax.ShapeDtypeStruct((n, EMB), jnp.float32),
        ],
    )(x, w3, bias)


def _tc_mid(parts, root, g, b, w3, bias):
    n = root.shape[0]

    def body(p_ref, r_ref, g_ref, b_ref, w_ref, bias_ref, y_ref, ro_ref):
        x = p_ref[0] + p_ref[1] + r_ref[...]
        x = jnp.maximum(x, 0.0)
        mu = jnp.mean(x, axis=-1, keepdims=True)
        var = jnp.mean((x - mu) ** 2, axis=-1, keepdims=True)
        x = (x - mu) / jnp.sqrt(var + 1e-5) * g_ref[...] + b_ref[...]
        y_ref[0] = jnp.dot(x, w_ref[0], preferred_element_type=jnp.float32)
        y_ref[1] = jnp.dot(x, w_ref[1], preferred_element_type=jnp.float32)
        ro_ref[...] = (jnp.dot(x, w_ref[2], preferred_element_type=jnp.float32)
                       + bias_ref[...])

    return pl.pallas_call(
        body,
        grid=(n // _TC_BLK,),
        in_specs=[
            pl.BlockSpec((2, _TC_BLK, EMB), lambda i: (0, i, 0)),
            pl.BlockSpec((_TC_BLK, EMB), lambda i: (i, 0)),
            pl.BlockSpec((1, EMB), lambda i: (0, 0)),
            pl.BlockSpec((1, EMB), lambda i: (0, 0)),
            pl.BlockSpec((3, EMB, EMB), lambda i: (0, 0, 0)),
            pl.BlockSpec((1, EMB), lambda i: (0, 0)),
        ],
        out_specs=[
            pl.BlockSpec((2, _TC_BLK, EMB), lambda i: (0, i, 0)),
            pl.BlockSpec((_TC_BLK, EMB), lambda i: (i, 0)),
        ],
        out_shape=[
            jax.ShapeDtypeStruct((2, n, EMB), jnp.float32),
            jax.ShapeDtypeStruct((n, EMB), jnp.float32),
        ],
    )(parts, root, g, b, w3, bias)


def _tc_combine(parts, root):
    n = root.shape[0]

    def body(p_ref, r_ref, o_ref):
        o_ref[...] = p_ref[0] + p_ref[1] + r_ref[...]

    return pl.pallas_call(
        body,
        grid=(n // _TC_BLK,),
        in_specs=[
            pl.BlockSpec((2, _TC_BLK, EMB), lambda i: (0, i, 0)),
            pl.BlockSpec((_TC_BLK, EMB), lambda i: (i, 0)),
        ],
        out_specs=pl.BlockSpec((_TC_BLK, EMB), lambda i: (i, 0)),
        out_shape=jax.ShapeDtypeStruct((n, EMB), jnp.float32),
    )(parts, root)


_HEAD_BLK = 1024


def _tc_head(gu, gi, w0a, w0b, b0, w1, b1, w2, b2, w3, b3, wout_a, wout_b, bo):
    batch = gu.shape[0]

    def body(gu_ref, gi_ref, w0a_ref, w0b_ref, b0_ref, w1_ref, b1_ref,
             w2_ref, b2_ref, w3_ref, b3_ref, wa_ref, wb_ref, bo_ref, o_ref):
        gu_b = gu_ref[...]
        gi_b = gi_ref[...]
        nu = jnp.sqrt(jnp.sum(gu_b * gu_b, axis=-1, keepdims=True))
        ni = jnp.sqrt(jnp.sum(gi_b * gi_b, axis=-1, keepdims=True))
        gmf = (gu_b / jnp.maximum(nu, 1e-12)) * (gi_b / jnp.maximum(ni, 1e-12))
        h = jnp.dot(gu_b, w0a_ref[...], preferred_element_type=jnp.float32)
        h = h + jnp.dot(gi_b, w0b_ref[...], preferred_element_type=jnp.float32)
        h = jnp.maximum(h + b0_ref[...], 0.0)
        h = jnp.maximum(
            jnp.dot(h, w1_ref[...], preferred_element_type=jnp.float32)
            + b1_ref[...], 0.0)
        h = jnp.maximum(
            jnp.dot(h, w2_ref[...], preferred_element_type=jnp.float32)
            + b2_ref[...], 0.0)
        h = jnp.maximum(
            jnp.dot(h, w3_ref[...], preferred_element_type=jnp.float32)
            + b3_ref[...], 0.0)
        s = (jnp.sum(gmf * wa_ref[...], axis=-1)
             + jnp.sum(h * wb_ref[...], axis=-1) + bo_ref[0, 0])
        o_ref[...] = s

    full = lambda shape: pl.BlockSpec(shape, lambda i: tuple(0 for _ in shape))
    return pl.pallas_call(
        body,
        grid=(batch // _HEAD_BLK,),
        in_specs=[
            pl.BlockSpec((_HEAD_BLK, EMB), lambda i: (i, 0)),
            pl.BlockSpec((_HEAD_BLK, EMB), lambda i: (i, 0)),
            full((EMB, 256)), full((EMB, 256)), full((1, 256)),
            full((256, 128)), full((1, 128)),
            full((128, 64)), full((1, 64)),
            full((64, 32)), full((1, 32)),
            full((1, EMB)), full((1, 32)), full((1, 1)),
        ],
        out_specs=pl.BlockSpec((_HEAD_BLK,), lambda i: (i,)),
        out_shape=jax.ShapeDtypeStruct((batch,), jnp.float32),
    )(gu, gi, w0a, w0b, b0, w1, b1, w2, b2, w3, b3, wout_a, wout_b, bo)


def kernel(user_indices, item_indices, edge_index, edge_type, edge_weight, emb,
           relW0, rootW0, bconv0, relW1, rootW1, bconv1, relW2, rootW2, bconv2,
           g1, b1, g2, b2,
           mlpW0, mlpb0, mlpW1, mlpb1, mlpW2, mlpb2, mlpW3, mlpb3,
           Wout, bout):
    n_nodes = emb.shape[0]
    n_edges = edge_weight.shape[0]

    src = edge_index[0].astype(jnp.int32)
    dst = edge_index[1].astype(jnp.int32)
    comb = edge_type.astype(jnp.int32) * n_nodes + src

    per_op = NW * CHUNK
    n_chunks = -(-n_edges // per_op)
    n_chunks = -(-n_chunks // NBUF) * NBUF
    pad = n_chunks * per_op - n_edges
    n_pad = -(-n_nodes // (NS * CHUNK)) * NS * CHUNK
    idx3 = jnp.pad(comb, (0, pad)).reshape(NW, n_chunks, CHUNK)
    wgt3 = jnp.pad(edge_weight, (0, pad)).reshape(NW, n_chunks, CHUNK)
    dst3 = jnp.pad(dst, (0, pad)).reshape(NW, n_chunks, CHUNK)

    layers = [
        (relW0, rootW0, bconv0, (g1, b1)),
        (relW1, rootW1, bconv1, (g2, b2)),
        (relW2, rootW2, bconv2, None),
    ]

    parts = root = None
    for li, (relw, rootw, bconv, ln) in enumerate(layers):
        w3 = jnp.concatenate([relw, rootw[None]], axis=0)
        if li == 0:
            y2, root = _tc_first(emb, w3, bconv[None])
        else:
            g, b = layers[li - 1][3]
            y2, root = _tc_mid(parts, root, g[None], b[None], w3, bconv[None])
        parts_flat = _sc_aggregate(y2.reshape(2 * n_nodes, EMB),
                                   idx3, wgt3, dst3, n_pad, n_chunks)
        parts = parts_flat.reshape(NC, n_pad, EMB)

    x3 = _tc_combine(parts, root)

    batch = user_indices.shape[0]
    all_idx = jnp.concatenate([user_indices, item_indices]).astype(jnp.int32)
    rows_per_tile = (2 * batch) // NW
    gathered = _sc_gather_rows(x3, all_idx.reshape(NW, rows_per_tile // CHUNK,
                                                   CHUNK), rows_per_tile)
    gu = gathered[:batch]
    gi = gathered[batch:]

    score = _tc_head(
        gu, gi,
        mlpW0[:EMB], mlpW0[EMB:], mlpb0[None],
        mlpW1, mlpb1[None], mlpW2, mlpb2[None], mlpW3, mlpb3[None],
        Wout[:EMB, 0][None], Wout[EMB:, 0][None], bout[None])
    return score


# speedup vs baseline: 1.8689x; 1.8689x over previous
import functools

import jax
import jax.numpy as jnp
from jax import lax
from jax.experimental import pallas as pl
from jax.experimental.pallas import tpu as pltpu
from jax.experimental.pallas import tpu_sc as plsc

NC, NS, LANES = 2, 16, 16
NW = NC * NS
EMB = 128
CHUNK = 128
FSTEPS = EMB // LANES
NBUF = 3


def _sc_mesh():
    return plsc.VectorSubcoreMesh(core_axis_name="c", subcore_axis_name="s",
                                  num_cores=NC, num_subcores=NS)


def _sc_aggregate(y, idx3, wgt3, dst3, n_pad, n_chunks):
    n_half = n_pad // 2
    acc_rows = n_half + CHUNK
    zslice = acc_rows // NS
    wslice = n_half // NS
    pn = n_chunks * CHUNK

    @functools.partial(
        pl.kernel,
        out_type=jax.ShapeDtypeStruct((NC * n_pad, EMB), jnp.float32),
        mesh=_sc_mesh(),
        scratch_types=[
            pltpu.VMEM((8, CHUNK), jnp.int32),
            pltpu.VMEM((8, CHUNK), jnp.float32),
            pltpu.VMEM((8, CHUNK), jnp.int32),
            pltpu.VMEM((pn + LANES,), jnp.int32),
            pltpu.VMEM((pn + LANES,), jnp.float32),
            pltpu.VMEM((pn + LANES,), jnp.int32),
            [pltpu.VMEM((CHUNK,), jnp.int32) for _ in range(2)],
            [pltpu.VMEM((CHUNK, EMB), jnp.float32) for _ in range(2)],
            pltpu.VMEM_SHARED((acc_rows, EMB), jnp.float32),
            [pltpu.SemaphoreType.DMA for _ in range(2)],
        ],
        compiler_params=pltpu.CompilerParams(needs_layout_passes=False),
    )
    def k(y_hbm, idx_hbm, wgt_hbm, dst_hbm, out_hbm,
          idx_st, wgt_st, dst_st, pidx, pwgt, pdst, dst_locs, bufs, acc,
          sem_g):
        cid = lax.axis_index("c")
        sid = lax.axis_index("s")
        tid = cid * NS + sid

        zeros = jnp.zeros((LANES,), jnp.float32)

        def part_body(s, carry):
            c0, cb = carry
            row = pl.ds(s * 8, 8)
            pltpu.sync_copy(idx_hbm.at[tid, row], idx_st)
            pltpu.sync_copy(wgt_hbm.at[tid, row], wgt_st)
            pltpu.sync_copy(dst_hbm.at[tid, row], dst_st)
            for r in range(8):
                for q in range(CHUNK // LANES):
                    sl = pl.ds(q * LANES, LANES)
                    d = dst_st[r, sl]
                    x = idx_st[r, sl]
                    w = wgt_st[r, sl]
                    m = d < n_half
                    pc0 = plsc.all_reduce_population_count(m)[0]
                    cb2 = cb - (LANES - pc0)
                    plsc.store_compressed(pidx.at[pl.ds(c0, LANES)], x, mask=m)
                    plsc.store_compressed(pwgt.at[pl.ds(c0, LANES)], w, mask=m)
                    plsc.store_compressed(pdst.at[pl.ds(c0, LANES)], d, mask=m)
                    mn = jnp.logical_not(m)
                    plsc.store_compressed(pidx.at[pl.ds(cb2, LANES)], x,
                                          mask=mn)
                    plsc.store_compressed(pwgt.at[pl.ds(cb2, LANES)], w,
                                          mask=mn)
                    plsc.store_compressed(pdst.at[pl.ds(cb2, LANES)], d,
                                          mask=mn)
                    c0 = c0 + pc0
                    cb = cb2
            return c0, cb

        n0, _ = pl.loop(0, n_chunks // 8,
                        init_carry=(jnp.int32(0), jnp.int32(pn)))(part_body)

        def start_gather(b, j):
            pltpu.async_copy(y_hbm.at[pidx.at[pl.ds(j * CHUNK, CHUNK)]],
                             bufs[b], sem_g[b])

        def wait_gather(b):
            pltpu.make_async_copy(y_hbm.at[pidx.at[pl.ds(0, CHUNK)]],
                                  bufs[b], sem_g[b]).wait()

        def scale(b, j):
            buf = bufs[b]
            base = j * CHUNK

            @pl.loop(0, CHUNK)
            def _(e):
                w = plsc.load_gather(
                    pwgt, [jnp.full((LANES,), base + e, jnp.int32)])
                for f in range(FSTEPS):
                    sl = pl.ds(f * LANES, LANES)
                    buf[e, sl] = buf[e, sl] * w

        lastc = n_chunks - 1

        for h in (0, 1):
            plsc.subcore_barrier()

            @pl.loop(0, CHUNK)
            def _(e):
                for f in range(FSTEPS):
                    bufs[0][e, pl.ds(f * LANES, LANES)] = zeros

            z0 = sid * zslice
            for off, size in ((0, CHUNK), (CHUNK, CHUNK),
                              (2 * CHUNK, zslice - 2 * CHUNK)):
                pltpu.sync_copy(bufs[0].at[pl.ds(0, size)],
                                acc.at[pl.ds(z0 + off, size)])
            plsc.subcore_barrier()

            def make_dst(b, j):
                base = j * CHUNK
                for q in range(CHUNK // LANES):
                    d = pdst[pl.ds(base + q * LANES, LANES)]
                    local = d - h * n_half
                    ok = (local >= 0) & (local < n_half)
                    dst_locs[b][pl.ds(q * LANES, LANES)] = jnp.where(
                        ok, local, n_half + (d & (CHUNK - 1)))

            def process(b, j):
                scale(b, j)
                make_dst(b, j)
                pltpu.sync_copy(bufs[b], acc.at[dst_locs[b]], add=True)

            if h == 0:
                lo = jnp.int32(0)
                hi = (n0 + CHUNK - 1) // CHUNK
            else:
                lo = n0 // CHUNK
                hi = jnp.int32(n_chunks)
            nsteps = (hi - lo + 1) // 2

            start_gather(0, jnp.minimum(lo, lastc))

            @pl.loop(0, nsteps)
            def _(g):
                j = lo + 2 * g
                wait_gather(0)
                start_gather(1, jnp.minimum(j + 1, lastc))
                process(0, j)
                wait_gather(1)
                start_gather(0, jnp.minimum(j + 2, lastc))

                @pl.when(j + 1 < hi)
                def _():
                    process(1, j + 1)

            wait_gather(0)
            plsc.subcore_barrier()

            for off, size in ((0, CHUNK), (CHUNK, CHUNK),
                              (2 * CHUNK, wslice - 2 * CHUNK)):
                r0 = sid * wslice + off
                pltpu.sync_copy(acc.at[pl.ds(r0, size)],
                                bufs[0].at[pl.ds(0, size)])
                pltpu.sync_copy(
                    bufs[0].at[pl.ds(0, size)],
                    out_hbm.at[pl.ds(cid * n_pad + h * n_half + r0, size)])

    return k(y, idx3, wgt3, dst3)


def _sc_gather_rows(x, idx3, n_rows_per_tile):
    nch = n_rows_per_tile // CHUNK
    n_out = NW * n_rows_per_tile

    @functools.partial(
        pl.kernel,
        out_type=jax.ShapeDtypeStruct((n_out, EMB), jnp.float32),
        mesh=_sc_mesh(),
        scratch_types=[
            pltpu.VMEM((nch, CHUNK), jnp.int32),
            pltpu.VMEM((CHUNK, EMB), jnp.float32),
            pltpu.VMEM((CHUNK, EMB), jnp.float32),
            pltpu.SemaphoreType.DMA,
            pltpu.SemaphoreType.DMA,
        ],
    )
    def k(x_hbm, idx_hbm, out_hbm, idx_v, buf_a, buf_b, sem_a, sem_b):
        cid = lax.axis_index("c")
        sid = lax.axis_index("s")
        tid = cid * NS + sid
        base = tid * n_rows_per_tile

        pltpu.sync_copy(idx_hbm.at[tid], idx_v)

        def start_gather(buf, sem, j):
            pltpu.async_copy(x_hbm.at[idx_v.at[j]], buf, sem)

        def wait_gather(buf, sem):
            pltpu.make_async_copy(x_hbm.at[idx_v.at[0]], buf, sem).wait()

        last = nch - 1
        start_gather(buf_a, sem_a, 0)

        @pl.loop(0, nch, step=2)
        def _(j):
            wait_gather(buf_a, sem_a)
            start_gather(buf_b, sem_b, j + 1)
            pltpu.sync_copy(buf_a, out_hbm.at[pl.ds(base + j * CHUNK, CHUNK)])
            wait_gather(buf_b, sem_b)
            start_gather(buf_a, sem_a, jnp.minimum(j + 2, last))
            pltpu.sync_copy(buf_b,
                            out_hbm.at[pl.ds(base + (j + 1) * CHUNK, CHUNK)])

        wait_gather(buf_a, sem_a)

    return k(x, idx3)


_TC_BLK = 1000


def _tc_first(x, w3, bias):
    n = x.shape[0]

    def body(x_ref, w_ref, b_ref, y_ref, r_ref):
        xb = x_ref[...]
        y_ref[0] = jnp.dot(xb, w_ref[0], preferred_element_type=jnp.float32)
        y_ref[1] = jnp.dot(xb, w_ref[1], preferred_element_type=jnp.float32)
        r_ref[...] = (jnp.dot(xb, w_ref[2], preferred_element_type=jnp.float32)
                      + b_ref[...])

    return pl.pallas_call(
        body,
        grid=(n // _TC_BLK,),
        in_specs=[
            pl.BlockSpec((_TC_BLK, EMB), lambda i: (i, 0)),
            pl.BlockSpec((3, EMB, EMB), lambda i: (0, 0, 0)),
            pl.BlockSpec((1, EMB), lambda i: (0, 0)),
        ],
        out_specs=[
            pl.BlockSpec((2, _TC_BLK, EMB), lambda i: (0, i, 0)),
            pl.BlockSpec((_TC_BLK, EMB), lambda i: (i, 0)),
        ],
        out_shape=[
            jax.ShapeDtypeStruct((2, n, EMB), jnp.float32),
            jax.ShapeDtypeStruct((n, EMB), jnp.float32),
        ],
    )(x, w3, bias)


def _tc_mid(parts, root, g, b, w3, bias):
    n = root.shape[0]

    def body(p_ref, r_ref, g_ref, b_ref, w_ref, bias_ref, y_ref, ro_ref):
        x = p_ref[0] + p_ref[1] + r_ref[...]
        x = jnp.maximum(x, 0.0)
        mu = jnp.mean(x, axis=-1, keepdims=True)
        var = jnp.mean((x - mu) ** 2, axis=-1, keepdims=True)
        x = (x - mu) / jnp.sqrt(var + 1e-5) * g_ref[...] + b_ref[...]
        y_ref[0] = jnp.dot(x, w_ref[0], preferred_element_type=jnp.float32)
        y_ref[1] = jnp.dot(x, w_ref[1], preferred_element_type=jnp.float32)
        ro_ref[...] = (jnp.dot(x, w_ref[2], preferred_element_type=jnp.float32)
                       + bias_ref[...])

    return pl.pallas_call(
        body,
        grid=(n // _TC_BLK,),
        in_specs=[
            pl.BlockSpec((2, _TC_BLK, EMB), lambda i: (0, i, 0)),
            pl.BlockSpec((_TC_BLK, EMB), lambda i: (i, 0)),
            pl.BlockSpec((1, EMB), lambda i: (0, 0)),
            pl.BlockSpec((1, EMB), lambda i: (0, 0)),
            pl.BlockSpec((3, EMB, EMB), lambda i: (0, 0, 0)),
            pl.BlockSpec((1, EMB), lambda i: (0, 0)),
        ],
        out_specs=[
            pl.BlockSpec((2, _TC_BLK, EMB), lambda i: (0, i, 0)),
            pl.BlockSpec((_TC_BLK, EMB), lambda i: (i, 0)),
        ],
        out_shape=[
            jax.ShapeDtypeStruct((2, n, EMB), jnp.float32),
            jax.ShapeDtypeStruct((n, EMB), jnp.float32),
        ],
    )(parts, root, g, b, w3, bias)


def _tc_combine(parts, root):
    n = root.shape[0]

    def body(p_ref, r_ref, o_ref):
        o_ref[...] = p_ref[0] + p_ref[1] + r_ref[...]

    return pl.pallas_call(
        body,
        grid=(n // _TC_BLK,),
        in_specs=[
            pl.BlockSpec((2, _TC_BLK, EMB), lambda i: (0, i, 0)),
            pl.BlockSpec((_TC_BLK, EMB), lambda i: (i, 0)),
        ],
        out_specs=pl.BlockSpec((_TC_BLK, EMB), lambda i: (i, 0)),
        out_shape=jax.ShapeDtypeStruct((n, EMB), jnp.float32),
    )(parts, root)


_HEAD_BLK = 1024


def _tc_head(gu, gi, w0a, w0b, b0, w1, b1, w2, b2, w3, b3, wout_a, wout_b, bo):
    batch = gu.shape[0]

    def body(gu_ref, gi_ref, w0a_ref, w0b_ref, b0_ref, w1_ref, b1_ref,
             w2_ref, b2_ref, w3_ref, b3_ref, wa_ref, wb_ref, bo_ref, o_ref):
        gu_b = gu_ref[...]
        gi_b = gi_ref[...]
        nu = jnp.sqrt(jnp.sum(gu_b * gu_b, axis=-1, keepdims=True))
        ni = jnp.sqrt(jnp.sum(gi_b * gi_b, axis=-1, keepdims=True))
        gmf = (gu_b / jnp.maximum(nu, 1e-12)) * (gi_b / jnp.maximum(ni, 1e-12))
        h = jnp.dot(gu_b, w0a_ref[...], preferred_element_type=jnp.float32)
        h = h + jnp.dot(gi_b, w0b_ref[...], preferred_element_type=jnp.float32)
        h = jnp.maximum(h + b0_ref[...], 0.0)
        h = jnp.maximum(
            jnp.dot(h, w1_ref[...], preferred_element_type=jnp.float32)
            + b1_ref[...], 0.0)
        h = jnp.maximum(
            jnp.dot(h, w2_ref[...], preferred_element_type=jnp.float32)
            + b2_ref[...], 0.0)
        h = jnp.maximum(
            jnp.dot(h, w3_ref[...], preferred_element_type=jnp.float32)
            + b3_ref[...], 0.0)
        s = (jnp.sum(gmf * wa_ref[...], axis=-1)
             + jnp.sum(h * wb_ref[...], axis=-1) + bo_ref[0, 0])
        o_ref[...] = s

    full = lambda shape: pl.BlockSpec(shape, lambda i: tuple(0 for _ in shape))
    return pl.pallas_call(
        body,
        grid=(batch // _HEAD_BLK,),
        in_specs=[
            pl.BlockSpec((_HEAD_BLK, EMB), lambda i: (i, 0)),
            pl.BlockSpec((_HEAD_BLK, EMB), lambda i: (i, 0)),
            full((EMB, 256)), full((EMB, 256)), full((1, 256)),
            full((256, 128)), full((1, 128)),
            full((128, 64)), full((1, 64)),
            full((64, 32)), full((1, 32)),
            full((1, EMB)), full((1, 32)), full((1, 1)),
        ],
        out_specs=pl.BlockSpec((_HEAD_BLK,), lambda i: (i,)),
        out_shape=jax.ShapeDtypeStruct((batch,), jnp.float32),
    )(gu, gi, w0a, w0b, b0, w1, b1, w2, b2, w3, b3, wout_a, wout_b, bo)


def kernel(user_indices, item_indices, edge_index, edge_type, edge_weight, emb,
           relW0, rootW0, bconv0, relW1, rootW1, bconv1, relW2, rootW2, bconv2,
           g1, b1, g2, b2,
           mlpW0, mlpb0, mlpW1, mlpb1, mlpW2, mlpb2, mlpW3, mlpb3,
           Wout, bout):
    n_nodes = emb.shape[0]
    n_edges = edge_weight.shape[0]

    src = edge_index[0].astype(jnp.int32)
    dst = edge_index[1].astype(jnp.int32)
    comb = edge_type.astype(jnp.int32) * n_nodes + src

    per_op = NW * CHUNK
    n_chunks = -(-n_edges // per_op)
    n_chunks = -(-n_chunks // 8) * 8
    pad = n_chunks * per_op - n_edges
    n_pad = -(-n_nodes // (NS * CHUNK)) * NS * CHUNK
    idx3 = jnp.pad(comb, (0, pad)).reshape(NW, n_chunks, CHUNK)
    wgt3 = jnp.pad(edge_weight, (0, pad)).reshape(NW, n_chunks, CHUNK)
    dst3 = jnp.pad(dst, (0, pad)).reshape(NW, n_chunks, CHUNK)

    layers = [
        (relW0, rootW0, bconv0, (g1, b1)),
        (relW1, rootW1, bconv1, (g2, b2)),
        (relW2, rootW2, bconv2, None),
    ]

    parts = root = None
    for li, (relw, rootw, bconv, ln) in enumerate(layers):
        w3 = jnp.concatenate([relw, rootw[None]], axis=0)
        if li == 0:
            y2, root = _tc_first(emb, w3, bconv[None])
        else:
            g, b = layers[li - 1][3]
            y2, root = _tc_mid(parts, root, g[None], b[None], w3, bconv[None])
        parts_flat = _sc_aggregate(y2.reshape(2 * n_nodes, EMB),
                                   idx3, wgt3, dst3, n_pad, n_chunks)
        parts = parts_flat.reshape(NC, n_pad, EMB)

    x3 = _tc_combine(parts, root)

    batch = user_indices.shape[0]
    all_idx = jnp.concatenate([user_indices, item_indices]).astype(jnp.int32)
    rows_per_tile = (2 * batch) // NW
    gathered = _sc_gather_rows(x3, all_idx.reshape(NW, rows_per_tile // CHUNK,
                                                   CHUNK), rows_per_tile)
    gu = gathered[:batch]
    gi = gathered[batch:]

    score = _tc_head(
        gu, gi,
        mlpW0[:EMB], mlpW0[EMB:], mlpb0[None],
        mlpW1, mlpb1[None], mlpW2, mlpb2[None], mlpW3, mlpb3[None],
        Wout[:EMB, 0][None], Wout[EMB:, 0][None], bout[None])
    return score
